# Initial kernel scaffold; baseline (speedup 1.0000x reference)
import jax
import jax.numpy as jnp
from jax.experimental import pallas as pl


def kernel(x, edge_attr, W1a, b1a, W2a, b2a, Wc2, bc2, W1b, b1b, W2b, b2b, Wc4, bc4, edge_index):
    raise NotImplementedError("write your pallas kernel here")


# speedup vs baseline: 1.4565x; 1.4565x over previous
import functools

import jax
import jax.numpy as jnp
from jax import lax
from jax.experimental import pallas as pl
from jax.experimental.pallas import tpu as pltpu

_OBS = 0.5
_ATT = 0.3
_COMM = 0.7

_EB = 8000


def _min_body(xi_ref, xj_ref, out_ref):
    i = pl.program_id(0)
    ts = xj_ref[:, 0:1]
    td = xi_ref[:, 0:1]
    inf = jnp.float32(jnp.inf)
    a = jnp.min(ts)
    b = jnp.min(jnp.where(td == 1.0, ts, inf))
    c = jnp.min(jnp.where(td == 2.0, ts, inf))
    row = lax.broadcasted_iota(jnp.int32, (8, 128), 0)
    vals = jnp.where(row == 0, a, jnp.where(row == 1, b, jnp.where(row == 2, c, inf)))

    @pl.when(i == 0)
    def _():
        out_ref[...] = vals

    @pl.when(i > 0)
    def _():
        out_ref[...] = jnp.minimum(out_ref[...], vals)


def _edge1_body(mins_ref, xi_ref, xj_ref, ea_ref, w1i_ref, w1j_ref, w1e_ref,
                b1_ref, w2_ref, b2_ref, out_ref):
    xi = xi_ref[...]
    xj = xj_ref[...]
    ea = ea_ref[...]
    h = jnp.maximum(
        xi @ w1i_ref[...] + xj @ w1j_ref[...] + ea @ w1e_ref[...] + b1_ref[...],
        0.0)
    msg = h @ w2_ref[...] + b2_ref[...]
    mins = mins_ref[...]
    inf = jnp.float32(jnp.inf)

    def thr(m):
        return jnp.where(m == 0.0, _OBS, jnp.where(m == 1.0, _ATT, inf))

    thr_a = thr(mins[0:1, 0:1])
    thr_b = thr(mins[1:2, 0:1])
    thr_c = thr(mins[2:3, 0:1])
    dist = ea[:, 0:1]
    td = xi[:, 0:1]
    m_a = (dist < thr_a).astype(msg.dtype)
    thr_x = jnp.where(td == 1.0, thr_b, thr_c)
    m_x = (dist < thr_x).astype(msg.dtype)
    out_ref[...] = jnp.concatenate([msg * m_a, msg * m_x], axis=1)


def _node1_body(x_ref, a0_ref, a1_ref, wcx_ref, wca_ref, wcb_ref, wcc_ref,
                bc_ref, wix_ref, wih_ref, wjx_ref, wjh_ref, p2i_ref, p2j_ref):
    x = x_ref[...]
    acc = a0_ref[...] + a1_ref[...]
    agent = acc[:, 0:16]
    extra = acc[:, 16:32]
    t = x[:, 0:1]
    m1 = (t == 1.0).astype(x.dtype)
    m2 = (t == 2.0).astype(x.dtype)
    rx = jnp.maximum(x, 0.0)
    ra = jnp.maximum(agent, 0.0)
    re = jnp.maximum(extra, 0.0)
    h = (rx @ wcx_ref[...] + ra @ wca_ref[...] + (re * m1) @ wcb_ref[...]
         + (re * m2) @ wcc_ref[...] + bc_ref[...])
    p2i_ref[...] = x @ wix_ref[...] + h @ wih_ref[...]
    p2j_ref[...] = x @ wjx_ref[...] + h @ wjh_ref[...]


def _edge2_body(acc_ref, ea_ref, w1e_ref, b1_ref, w2_ref, b2_ref, out_ref):
    ea = ea_ref[...]
    pre = acc_ref[...] + ea @ w1e_ref[...] + b1_ref[...]
    msg = jnp.maximum(pre, 0.0) @ w2_ref[...] + b2_ref[...]
    m = (ea[:, 0:1] < _COMM).astype(msg.dtype)
    out_ref[...] = msg * m


def _node2_body(x_ref, g0_ref, g1_ref, wcx_ref, wca_ref, bc_ref, out_ref):
    x = x_ref[...]
    t = x[:, 0:1]
    ag = (g0_ref[...] + g1_ref[...]) * (t == 0.0).astype(x.dtype)
    out_ref[...] = (jnp.maximum(x, 0.0) @ wcx_ref[...]
                    + jnp.maximum(ag, 0.0) @ wca_ref[...] + bc_ref[...])


def _full(shape):
    return pl.BlockSpec(shape, lambda *_: tuple(0 for _ in shape))


def _eblk(shape):
    return pl.BlockSpec(shape, lambda i: (i, 0))


def _tc_mins(xi, xj, n_e):
    grid = (n_e // _EB,)
    return pl.pallas_call(
        _min_body,
        grid=grid,
        in_specs=[_eblk((_EB, 8)), _eblk((_EB, 8))],
        out_specs=_full((8, 128)),
        out_shape=jax.ShapeDtypeStruct((8, 128), jnp.float32),
    )(xi, xj)


def _tc_edge1(mins, xi, xj, ea, w1i, w1j, w1e, b1, w2, b2, n_e):
    grid = (n_e // _EB,)
    return pl.pallas_call(
        _edge1_body,
        grid=grid,
        in_specs=[_full((8, 128)), _eblk((_EB, 8)), _eblk((_EB, 8)),
                  _eblk((_EB, 8)), _full((8, 32)), _full((8, 32)),
                  _full((8, 32)), _full((1, 32)), _full((32, 16)),
                  _full((1, 16))],
        out_specs=_eblk((_EB, 32)),
        out_shape=jax.ShapeDtypeStruct((n_e, 32), jnp.float32),
    )(mins, xi, xj, ea, w1i, w1j, w1e, b1, w2, b2)


def _tc_node1(x8, a0, a1, wcx, wca, wcb, wcc, bc, wix, wih, wjx, wjh, n):
    return pl.pallas_call(
        _node1_body,
        grid=(1,),
        in_specs=[_full((n, 8)), _full((n, 32)), _full((n, 32)),
                  _full((8, 16)), _full((16, 16)), _full((16, 16)),
                  _full((16, 16)), _full((1, 16)), _full((8, 32)),
                  _full((16, 32)), _full((8, 32)), _full((16, 32))],
        out_specs=[_full((n, 32)), _full((n, 32))],
        out_shape=[jax.ShapeDtypeStruct((n, 32), jnp.float32),
                   jax.ShapeDtypeStruct((n, 32), jnp.float32)],
    )(x8, a0, a1, wcx, wca, wcb, wcc, bc, wix, wih, wjx, wjh)


def _tc_edge2(acc2, ea, w1e, b1, w2, b2, n_e):
    grid = (n_e // _EB,)
    return pl.pallas_call(
        _edge2_body,
        grid=grid,
        in_specs=[_eblk((_EB, 32)), _eblk((_EB, 8)), _full((8, 32)),
                  _full((1, 32)), _full((32, 16)), _full((1, 16))],
        out_specs=_eblk((_EB, 16)),
        out_shape=jax.ShapeDtypeStruct((n_e, 16), jnp.float32),
    )(acc2, ea, w1e, b1, w2, b2)


def _tc_node2(x8, g0, g1, wcx, wca, bc, n):
    return pl.pallas_call(
        _node2_body,
        grid=(1,),
        in_specs=[_full((n, 8)), _full((n, 16)), _full((n, 16)),
                  _full((8, 16)), _full((16, 16)), _full((1, 16))],
        out_specs=_full((n, 16)),
        out_shape=jax.ShapeDtypeStruct((n, 16), jnp.float32),
    )(x8, g0, g1, wcx, wca, bc)


def kernel(x, edge_attr, W1a, b1a, W2a, b2a, Wc2, bc2, W1b, b1b, W2b, b2b,
           Wc4, bc4, edge_index):
    n = x.shape[0]
    n_e = edge_index.shape[1]
    src = edge_index[0]
    dst = edge_index[1]

    x8 = jnp.pad(x, ((0, 0), (0, 3)))
    ea8 = jnp.pad(edge_attr, ((0, 0), (0, 5)))

    z8 = jnp.zeros((8, 32), jnp.float32)
    w1i = z8.at[0:5].set(W1a[0:5])
    w1j = z8.at[0:5].set(W1a[5:10])
    w1e = z8.at[0:3].set(W1a[10:13])
    b1 = b1a.reshape(1, 32)
    b2 = b2a.reshape(1, 16)

    wcx = jnp.zeros((8, 16), jnp.float32).at[0:5].set(Wc2[0:5])
    wca = Wc2[5:21]
    wcb = Wc2[21:37]
    wcc = Wc2[37:53]
    bc = bc2.reshape(1, 16)

    wix = z8.at[0:5].set(W1b[0:5])
    wih = W1b[5:21]
    wjx = z8.at[0:5].set(W1b[21:26])
    wjh = W1b[26:42]
    w1be = z8.at[0:3].set(W1b[42:45])
    b1l2 = b1b.reshape(1, 32)
    b2l2 = b2b.reshape(1, 16)

    wc4x = jnp.zeros((8, 16), jnp.float32).at[0:5].set(Wc4[0:5])
    wc4a = Wc4[5:21]
    bc4r = bc4.reshape(1, 16)

    xi = x8[dst]
    xj = x8[src]

    mins = _tc_mins(xi, xj, n_e)
    msgax = _tc_edge1(mins, xi, xj, ea8, w1i, w1j, w1e, b1, W2a, b2, n_e)

    acc = jnp.zeros((n, 32), jnp.float32).at[dst].add(msgax)
    zacc = jnp.zeros((n, 32), jnp.float32)

    p2i, p2j = _tc_node1(x8, acc, zacc, wcx, wca, wcb, wcc, bc,
                         wix, wih, wjx, wjh, n)

    acc2 = p2i[dst] + p2j[src]

    msg2 = _tc_edge2(acc2, ea8, w1be, b1l2, W2b, b2l2, n_e)

    g = jnp.zeros((n, 16), jnp.float32).at[dst].add(msg2)
    zg = jnp.zeros((n, 16), jnp.float32)

    return _tc_node2(x8, g, zg, wc4x, wc4a, bc4r, n)


# speedup vs baseline: 4.2370x; 2.9089x over previous
import functools

import jax
import jax.numpy as jnp
from jax import lax
from jax.experimental import pallas as pl
from jax.experimental.pallas import tpu as pltpu
from jax.experimental.pallas import tpu_sc as plsc

_NC = 2
_NS = 16
_NW = _NC * _NS

_OBS = 0.5
_ATT = 0.3
_COMM = 0.7

_EB = 8000


_IB = 80
_KB = 25


def _sc_gather_pair(t_i, t_j, src, dst, w, kb):
    n_e = src.shape[0]
    per_w = n_e // _NW
    c = _IB * kb
    n_chunks = per_w // c
    src2 = src.reshape(n_e // _IB, _IB)
    dst2 = dst.reshape(n_e // _IB, _IB)
    mesh = plsc.VectorSubcoreMesh(core_axis_name="c", subcore_axis_name="s")

    @functools.partial(
        pl.kernel,
        out_type=[jax.ShapeDtypeStruct((n_e, w), jnp.float32),
                  jax.ShapeDtypeStruct((n_e, w), jnp.float32)],
        mesh=mesh,
        compiler_params=pltpu.CompilerParams(use_tc_tiling_on_sc=False),
        scratch_types=[pltpu.VMEM((kb, _IB), jnp.int32),
                       pltpu.VMEM((kb, _IB), jnp.int32),
                       pltpu.VMEM((c, w), jnp.float32),
                       pltpu.VMEM((c, w), jnp.float32),
                       pltpu.SemaphoreType.DMA,
                       pltpu.SemaphoreType.DMA],
    )
    def k(ti_hbm, tj_hbm, src_hbm, dst_hbm, xi_hbm, xj_hbm, dbuf, sbuf,
          xib, xjb, sem1, sem2):
        wid = lax.axis_index("s") * _NC + lax.axis_index("c")
        base = wid * per_w

        def chunk(kk, _):
            off = base + kk * c
            row = off // _IB
            pltpu.sync_copy(dst_hbm.at[pl.ds(row, kb)], dbuf)
            pltpu.sync_copy(src_hbm.at[pl.ds(row, kb)], sbuf)
            for j in range(kb):
                pltpu.async_copy(ti_hbm.at[dbuf.at[j]],
                                 xib.at[pl.ds(j * _IB, _IB)], sem1)
                pltpu.async_copy(tj_hbm.at[sbuf.at[j]],
                                 xjb.at[pl.ds(j * _IB, _IB)], sem2)
            for j in range(kb):
                pltpu.make_async_copy(ti_hbm.at[dbuf.at[j]],
                                      xib.at[pl.ds(j * _IB, _IB)], sem1).wait()
                pltpu.make_async_copy(tj_hbm.at[sbuf.at[j]],
                                      xjb.at[pl.ds(j * _IB, _IB)], sem2).wait()
            pltpu.sync_copy(xib, xi_hbm.at[pl.ds(off, c)])
            pltpu.sync_copy(xjb, xj_hbm.at[pl.ds(off, c)])
            return 0

        lax.fori_loop(0, n_chunks, chunk, 0)

    return k(t_i, t_j, src2, dst2)


def _sc_scatter_add(msg, dst, zeros, n, w):
    n_e = msg.shape[0]
    per_w = n_e // _NW
    c = _IB * _KB
    n_chunks = per_w // c
    rows = n // _NS
    dst2 = dst.reshape(n_e // _IB, _IB)
    mesh = plsc.VectorSubcoreMesh(core_axis_name="c", subcore_axis_name="s")

    @functools.partial(
        pl.kernel,
        out_type=jax.ShapeDtypeStruct((2, n, w), jnp.float32),
        mesh=mesh,
        compiler_params=pltpu.CompilerParams(use_tc_tiling_on_sc=False),
        scratch_types=[pltpu.VMEM((_KB, _IB), jnp.int32),
                       pltpu.VMEM((c, w), jnp.float32),
                       pltpu.VMEM_SHARED((n, w), jnp.float32)],
    )
    def k(msg_hbm, dst_hbm, zeros_hbm, accs_hbm, dbuf, mbuf, acc_sh):
        cid = lax.axis_index("c")
        sid = lax.axis_index("s")
        wid = sid * _NC + cid
        base = wid * per_w
        pltpu.sync_copy(zeros_hbm.at[pl.ds(sid * rows, rows)],
                        acc_sh.at[pl.ds(sid * rows, rows)])
        plsc.subcore_barrier()

        def chunk(kk, _):
            off = base + kk * c
            row = off // _IB
            pltpu.sync_copy(dst_hbm.at[pl.ds(row, _KB)], dbuf)
            pltpu.sync_copy(msg_hbm.at[pl.ds(off, c)], mbuf)
            for j in range(_KB):
                pltpu.sync_copy(mbuf.at[pl.ds(j * _IB, _IB)],
                                acc_sh.at[dbuf.at[j]], add=True)
            return 0

        lax.fori_loop(0, n_chunks, chunk, 0)
        plsc.subcore_barrier()
        pltpu.sync_copy(acc_sh.at[pl.ds(sid * rows, rows)],
                        accs_hbm.at[cid].at[pl.ds(sid * rows, rows)])

    return k(msg, dst2, zeros)


def _min_body(xi_ref, xj_ref, out_ref):
    i = pl.program_id(0)
    ts = xj_ref[:, 0:1]
    td = xi_ref[:, 0:1]
    inf = jnp.float32(jnp.inf)
    a = jnp.min(ts)
    b = jnp.min(jnp.where(td == 1.0, ts, inf))
    c = jnp.min(jnp.where(td == 2.0, ts, inf))
    row = lax.broadcasted_iota(jnp.int32, (8, 128), 0)
    vals = jnp.where(row == 0, a, jnp.where(row == 1, b, jnp.where(row == 2, c, inf)))

    @pl.when(i == 0)
    def _():
        out_ref[...] = vals

    @pl.when(i > 0)
    def _():
        out_ref[...] = jnp.minimum(out_ref[...], vals)


def _edge1_body(mins_ref, xi_ref, xj_ref, ea_ref, w1i_ref, w1j_ref, w1e_ref,
                b1_ref, w2_ref, b2_ref, out_ref):
    xi = xi_ref[...]
    xj = xj_ref[...]
    ea = ea_ref[...]
    h = jnp.maximum(
        xi @ w1i_ref[...] + xj @ w1j_ref[...] + ea @ w1e_ref[...] + b1_ref[...],
        0.0)
    msg = h @ w2_ref[...] + b2_ref[...]
    mins = mins_ref[...]
    inf = jnp.float32(jnp.inf)

    def thr(m):
        return jnp.where(m == 0.0, _OBS, jnp.where(m == 1.0, _ATT, inf))

    thr_a = thr(mins[0:1, 0:1])
    thr_b = thr(mins[1:2, 0:1])
    thr_c = thr(mins[2:3, 0:1])
    dist = ea[:, 0:1]
    td = xi[:, 0:1]
    m_a = (dist < thr_a).astype(msg.dtype)
    thr_x = jnp.where(td == 1.0, thr_b, thr_c)
    m_x = (dist < thr_x).astype(msg.dtype)
    out_ref[...] = jnp.concatenate([msg * m_a, msg * m_x], axis=1)


def _node1_body(x_ref, a0_ref, a1_ref, wcx_ref, wca_ref, wcb_ref, wcc_ref,
                bc_ref, wix_ref, wih_ref, wjx_ref, wjh_ref, p2i_ref, p2j_ref):
    x = x_ref[...]
    acc = a0_ref[...] + a1_ref[...]
    agent = acc[:, 0:16]
    extra = acc[:, 16:32]
    t = x[:, 0:1]
    m1 = (t == 1.0).astype(x.dtype)
    m2 = (t == 2.0).astype(x.dtype)
    rx = jnp.maximum(x, 0.0)
    ra = jnp.maximum(agent, 0.0)
    re = jnp.maximum(extra, 0.0)
    h = (rx @ wcx_ref[...] + ra @ wca_ref[...] + (re * m1) @ wcb_ref[...]
         + (re * m2) @ wcc_ref[...] + bc_ref[...])
    p2i_ref[...] = x @ wix_ref[...] + h @ wih_ref[...]
    p2j_ref[...] = x @ wjx_ref[...] + h @ wjh_ref[...]


def _edge2_body(gi_ref, gj_ref, ea_ref, w1e_ref, b1_ref, w2_ref, b2_ref,
                out_ref):
    ea = ea_ref[...]
    pre = gi_ref[...] + gj_ref[...] + ea @ w1e_ref[...] + b1_ref[...]
    msg = jnp.maximum(pre, 0.0) @ w2_ref[...] + b2_ref[...]
    m = (ea[:, 0:1] < _COMM).astype(msg.dtype)
    out_ref[...] = msg * m


def _node2_body(x_ref, g0_ref, g1_ref, wcx_ref, wca_ref, bc_ref, out_ref):
    x = x_ref[...]
    t = x[:, 0:1]
    ag = (g0_ref[...] + g1_ref[...]) * (t == 0.0).astype(x.dtype)
    out_ref[...] = (jnp.maximum(x, 0.0) @ wcx_ref[...]
                    + jnp.maximum(ag, 0.0) @ wca_ref[...] + bc_ref[...])


def _full(shape):
    return pl.BlockSpec(shape, lambda *_: tuple(0 for _ in shape))


def _eblk(shape):
    return pl.BlockSpec(shape, lambda i: (i, 0))


def _tc_mins(xi, xj, n_e):
    grid = (n_e // _EB,)
    return pl.pallas_call(
        _min_body,
        grid=grid,
        in_specs=[_eblk((_EB, 8)), _eblk((_EB, 8))],
        out_specs=_full((8, 128)),
        out_shape=jax.ShapeDtypeStruct((8, 128), jnp.float32),
    )(xi, xj)


def _tc_edge1(mins, xi, xj, ea, w1i, w1j, w1e, b1, w2, b2, n_e):
    grid = (n_e // _EB,)
    return pl.pallas_call(
        _edge1_body,
        grid=grid,
        in_specs=[_full((8, 128)), _eblk((_EB, 8)), _eblk((_EB, 8)),
                  _eblk((_EB, 8)), _full((8, 32)), _full((8, 32)),
                  _full((8, 32)), _full((1, 32)), _full((32, 16)),
                  _full((1, 16))],
        out_specs=_eblk((_EB, 32)),
        out_shape=jax.ShapeDtypeStruct((n_e, 32), jnp.float32),
    )(mins, xi, xj, ea, w1i, w1j, w1e, b1, w2, b2)


def _tc_node1(x8, a0, a1, wcx, wca, wcb, wcc, bc, wix, wih, wjx, wjh, n):
    return pl.pallas_call(
        _node1_body,
        grid=(1,),
        in_specs=[_full((n, 8)), _full((n, 32)), _full((n, 32)),
                  _full((8, 16)), _full((16, 16)), _full((16, 16)),
                  _full((16, 16)), _full((1, 16)), _full((8, 32)),
                  _full((16, 32)), _full((8, 32)), _full((16, 32))],
        out_specs=[_full((n, 32)), _full((n, 32))],
        out_shape=[jax.ShapeDtypeStruct((n, 32), jnp.float32),
                   jax.ShapeDtypeStruct((n, 32), jnp.float32)],
    )(x8, a0, a1, wcx, wca, wcb, wcc, bc, wix, wih, wjx, wjh)


def _tc_edge2(gi, gj, ea, w1e, b1, w2, b2, n_e):
    grid = (n_e // _EB,)
    return pl.pallas_call(
        _edge2_body,
        grid=grid,
        in_specs=[_eblk((_EB, 32)), _eblk((_EB, 32)), _eblk((_EB, 8)),
                  _full((8, 32)), _full((1, 32)), _full((32, 16)),
                  _full((1, 16))],
        out_specs=_eblk((_EB, 16)),
        out_shape=jax.ShapeDtypeStruct((n_e, 16), jnp.float32),
    )(gi, gj, ea, w1e, b1, w2, b2)


def _tc_node2(x8, g0, g1, wcx, wca, bc, n):
    return pl.pallas_call(
        _node2_body,
        grid=(1,),
        in_specs=[_full((n, 8)), _full((n, 16)), _full((n, 16)),
                  _full((8, 16)), _full((16, 16)), _full((1, 16))],
        out_specs=_full((n, 16)),
        out_shape=jax.ShapeDtypeStruct((n, 16), jnp.float32),
    )(x8, g0, g1, wcx, wca, bc)


def kernel(x, edge_attr, W1a, b1a, W2a, b2a, Wc2, bc2, W1b, b1b, W2b, b2b,
           Wc4, bc4, edge_index):
    n = x.shape[0]
    n_e = edge_index.shape[1]
    src = edge_index[0]
    dst = edge_index[1]

    x8 = jnp.pad(x, ((0, 0), (0, 3)))
    ea8 = jnp.pad(edge_attr, ((0, 0), (0, 5)))

    z8 = jnp.zeros((8, 32), jnp.float32)
    w1i = z8.at[0:5].set(W1a[0:5])
    w1j = z8.at[0:5].set(W1a[5:10])
    w1e = z8.at[0:3].set(W1a[10:13])
    b1 = b1a.reshape(1, 32)
    b2 = b2a.reshape(1, 16)

    wcx = jnp.zeros((8, 16), jnp.float32).at[0:5].set(Wc2[0:5])
    wca = Wc2[5:21]
    wcb = Wc2[21:37]
    wcc = Wc2[37:53]
    bc = bc2.reshape(1, 16)

    wix = z8.at[0:5].set(W1b[0:5])
    wih = W1b[5:21]
    wjx = z8.at[0:5].set(W1b[21:26])
    wjh = W1b[26:42]
    w1be = z8.at[0:3].set(W1b[42:45])
    b1l2 = b1b.reshape(1, 32)
    b2l2 = b2b.reshape(1, 16)

    wc4x = jnp.zeros((8, 16), jnp.float32).at[0:5].set(Wc4[0:5])
    wc4a = Wc4[5:21]
    bc4r = bc4.reshape(1, 16)

    xi, xj = _sc_gather_pair(x8, x8, src, dst, 8, _KB)

    mins = _tc_mins(xi, xj, n_e)
    msgax = _tc_edge1(mins, xi, xj, ea8, w1i, w1j, w1e, b1, W2a, b2, n_e)

    accs = _sc_scatter_add(msgax, dst, jnp.zeros((n, 32), jnp.float32), n, 32)

    p2i, p2j = _tc_node1(x8, accs[0], accs[1], wcx, wca, wcb, wcc, bc,
                         wix, wih, wjx, wjh, n)

    gi, gj = _sc_gather_pair(p2i, p2j, src, dst, 32, 10)

    msg2 = _tc_edge2(gi, gj, ea8, w1be, b1l2, W2b, b2l2, n_e)

    gs = _sc_scatter_add(msg2, dst, jnp.zeros((n, 16), jnp.float32), n, 16)

    return _tc_node2(x8, gs[0], gs[1], wc4x, wc4a, bc4r, n)


# speedup vs baseline: 6.5413x; 1.5439x over previous
import functools

import jax
import jax.numpy as jnp
from jax import lax
from jax.experimental import pallas as pl
from jax.experimental.pallas import tpu as pltpu
from jax.experimental.pallas import tpu_sc as plsc

_NC = 2
_NS = 16
_NW = _NC * _NS

_OBS = 0.5
_ATT = 0.3
_COMM = 0.7

_EB = 6400


_IB = 80
_KB = 25


def _sc_gather_pair(t_i, t_j, src, dst, w, kb):
    n_e = src.shape[0]
    per_w = n_e // _NW
    c = _IB * kb
    n_chunks = per_w // c
    src2 = src.reshape(n_e // _IB, _IB)
    dst2 = dst.reshape(n_e // _IB, _IB)
    mesh = plsc.VectorSubcoreMesh(core_axis_name="c", subcore_axis_name="s")

    @functools.partial(
        pl.kernel,
        out_type=[jax.ShapeDtypeStruct((n_e, w), jnp.float32),
                  jax.ShapeDtypeStruct((n_e, w), jnp.float32)],
        mesh=mesh,
        compiler_params=pltpu.CompilerParams(use_tc_tiling_on_sc=False),
        scratch_types=[pltpu.VMEM((kb, _IB), jnp.int32),
                       pltpu.VMEM((kb, _IB), jnp.int32),
                       pltpu.VMEM((c, w), jnp.float32),
                       pltpu.VMEM((c, w), jnp.float32),
                       pltpu.SemaphoreType.DMA,
                       pltpu.SemaphoreType.DMA],
    )
    def k(ti_hbm, tj_hbm, src_hbm, dst_hbm, xi_hbm, xj_hbm, dbuf, sbuf,
          xib, xjb, sem1, sem2):
        wid = lax.axis_index("s") * _NC + lax.axis_index("c")
        base = wid * per_w

        def chunk(kk, _):
            off = base + kk * c
            row = off // _IB
            pltpu.sync_copy(dst_hbm.at[pl.ds(row, kb)], dbuf)
            pltpu.sync_copy(src_hbm.at[pl.ds(row, kb)], sbuf)
            for j in range(kb):
                pltpu.async_copy(ti_hbm.at[dbuf.at[j]],
                                 xib.at[pl.ds(j * _IB, _IB)], sem1)
                pltpu.async_copy(tj_hbm.at[sbuf.at[j]],
                                 xjb.at[pl.ds(j * _IB, _IB)], sem2)
            for j in range(kb):
                pltpu.make_async_copy(ti_hbm.at[dbuf.at[j]],
                                      xib.at[pl.ds(j * _IB, _IB)], sem1).wait()
                pltpu.make_async_copy(tj_hbm.at[sbuf.at[j]],
                                      xjb.at[pl.ds(j * _IB, _IB)], sem2).wait()
            pltpu.sync_copy(xib, xi_hbm.at[pl.ds(off, c)])
            pltpu.sync_copy(xjb, xj_hbm.at[pl.ds(off, c)])
            return 0

        lax.fori_loop(0, n_chunks, chunk, 0)

    return k(t_i, t_j, src2, dst2)


def _sc_scatter_add(msg, dst, zeros, n, w):
    n_e = msg.shape[0]
    per_w = n_e // _NW
    c = _IB * _KB
    n_chunks = per_w // c
    rows = n // _NS
    dst2 = dst.reshape(n_e // _IB, _IB)
    mesh = plsc.VectorSubcoreMesh(core_axis_name="c", subcore_axis_name="s")

    @functools.partial(
        pl.kernel,
        out_type=jax.ShapeDtypeStruct((2, n, w), jnp.float32),
        mesh=mesh,
        compiler_params=pltpu.CompilerParams(use_tc_tiling_on_sc=False),
        scratch_types=[pltpu.VMEM((_KB, _IB), jnp.int32),
                       pltpu.VMEM((c, w), jnp.float32),
                       pltpu.VMEM_SHARED((n, w), jnp.float32)],
    )
    def k(msg_hbm, dst_hbm, zeros_hbm, accs_hbm, dbuf, mbuf, acc_sh):
        cid = lax.axis_index("c")
        sid = lax.axis_index("s")
        wid = sid * _NC + cid
        base = wid * per_w
        pltpu.sync_copy(zeros_hbm.at[pl.ds(sid * rows, rows)],
                        acc_sh.at[pl.ds(sid * rows, rows)])
        plsc.subcore_barrier()

        def chunk(kk, _):
            off = base + kk * c
            row = off // _IB
            pltpu.sync_copy(dst_hbm.at[pl.ds(row, _KB)], dbuf)
            pltpu.sync_copy(msg_hbm.at[pl.ds(off, c)], mbuf)
            for j in range(_KB):
                pltpu.sync_copy(mbuf.at[pl.ds(j * _IB, _IB)],
                                acc_sh.at[dbuf.at[j]], add=True)
            return 0

        lax.fori_loop(0, n_chunks, chunk, 0)
        plsc.subcore_barrier()
        pltpu.sync_copy(acc_sh.at[pl.ds(sid * rows, rows)],
                        accs_hbm.at[cid].at[pl.ds(sid * rows, rows)])

    return k(msg, dst2, zeros)


def _min_body(xi_ref, xj_ref, out_ref):
    i = pl.program_id(0)
    ts = xj_ref[:, 0:1]
    td = xi_ref[:, 0:1]
    inf = jnp.float32(jnp.inf)
    a = jnp.min(ts)
    b = jnp.min(jnp.where(td == 1.0, ts, inf))
    c = jnp.min(jnp.where(td == 2.0, ts, inf))
    row = lax.broadcasted_iota(jnp.int32, (8, 128), 0)
    vals = jnp.where(row == 0, a, jnp.where(row == 1, b, jnp.where(row == 2, c, inf)))

    @pl.when(i == 0)
    def _():
        out_ref[...] = vals

    @pl.when(i > 0)
    def _():
        out_ref[...] = jnp.minimum(out_ref[...], vals)


_EAT_DN = (((0,), (0,)), ((), ()))


def _dist_col(eat):
    sel = (lax.broadcasted_iota(jnp.int32, (8, 1), 0) == 0).astype(jnp.float32)
    return lax.dot_general(eat, sel, _EAT_DN,
                           precision=lax.Precision.HIGHEST)


def _edge1_body(mins_ref, xi_ref, xj_ref, eat_ref, w1i_ref, w1j_ref, w1e_ref,
                b1_ref, w2_ref, b2_ref, out_ref):
    xi = xi_ref[...]
    xj = xj_ref[...]
    eat = eat_ref[...]
    h = jnp.maximum(
        xi @ w1i_ref[...] + xj @ w1j_ref[...]
        + lax.dot_general(eat, w1e_ref[...], _EAT_DN) + b1_ref[...],
        0.0)
    msg = h @ w2_ref[...] + b2_ref[...]
    mins = mins_ref[...]
    inf = jnp.float32(jnp.inf)

    def thr(m):
        return jnp.where(m == 0.0, _OBS, jnp.where(m == 1.0, _ATT, inf))

    thr_a = thr(mins[0:1, 0:1])
    thr_b = thr(mins[1:2, 0:1])
    thr_c = thr(mins[2:3, 0:1])
    dist = _dist_col(eat)
    td = xi[:, 0:1]
    m_a = (dist < thr_a).astype(msg.dtype)
    thr_x = jnp.where(td == 1.0, thr_b, thr_c)
    m_x = (dist < thr_x).astype(msg.dtype)
    out_ref[...] = jnp.concatenate([msg * m_a, msg * m_x], axis=1)


def _node1_body(x_ref, a0_ref, a1_ref, wcx_ref, wca_ref, wcb_ref, wcc_ref,
                bc_ref, wix_ref, wih_ref, wjx_ref, wjh_ref, p2i_ref, p2j_ref):
    x = x_ref[...]
    acc = a0_ref[...] + a1_ref[...]
    agent = acc[:, 0:16]
    extra = acc[:, 16:32]
    t = x[:, 0:1]
    m1 = (t == 1.0).astype(x.dtype)
    m2 = (t == 2.0).astype(x.dtype)
    rx = jnp.maximum(x, 0.0)
    ra = jnp.maximum(agent, 0.0)
    re = jnp.maximum(extra, 0.0)
    h = (rx @ wcx_ref[...] + ra @ wca_ref[...] + (re * m1) @ wcb_ref[...]
         + (re * m2) @ wcc_ref[...] + bc_ref[...])
    p2i_ref[...] = x @ wix_ref[...] + h @ wih_ref[...]
    p2j_ref[...] = x @ wjx_ref[...] + h @ wjh_ref[...]


def _edge2_body(gi_ref, gj_ref, eat_ref, w1e_ref, b1_ref, w2_ref, b2_ref,
                out_ref):
    eat = eat_ref[...]
    pre = (gi_ref[...] + gj_ref[...]
           + lax.dot_general(eat, w1e_ref[...], _EAT_DN) + b1_ref[...])
    msg = jnp.maximum(pre, 0.0) @ w2_ref[...] + b2_ref[...]
    m = (_dist_col(eat) < _COMM).astype(msg.dtype)
    out_ref[...] = msg * m


def _node2_body(x_ref, g0_ref, g1_ref, wcx_ref, wca_ref, bc_ref, out_ref):
    x = x_ref[...]
    t = x[:, 0:1]
    ag = (g0_ref[...] + g1_ref[...]) * (t == 0.0).astype(x.dtype)
    out_ref[...] = (jnp.maximum(x, 0.0) @ wcx_ref[...]
                    + jnp.maximum(ag, 0.0) @ wca_ref[...] + bc_ref[...])


def _full(shape):
    return pl.BlockSpec(shape, lambda *_: tuple(0 for _ in shape))


def _eblk(shape):
    return pl.BlockSpec(shape, lambda i: (i, 0))


def _tc_mins(xi, xj, n_e):
    grid = (n_e // _EB,)
    return pl.pallas_call(
        _min_body,
        grid=grid,
        in_specs=[_eblk((_EB, 8)), _eblk((_EB, 8))],
        out_specs=_full((8, 128)),
        out_shape=jax.ShapeDtypeStruct((8, 128), jnp.float32),
    )(xi, xj)


def _teblk(shape):
    return pl.BlockSpec(shape, lambda i: (0, i))


def _tc_edge1(mins, xi, xj, eat, w1i, w1j, w1e, b1, w2, b2, n_e):
    grid = (n_e // _EB,)
    return pl.pallas_call(
        _edge1_body,
        grid=grid,
        in_specs=[_full((8, 128)), _eblk((_EB, 8)), _eblk((_EB, 8)),
                  _teblk((8, _EB)), _full((8, 32)), _full((8, 32)),
                  _full((8, 32)), _full((1, 32)), _full((32, 16)),
                  _full((1, 16))],
        out_specs=_eblk((_EB, 32)),
        out_shape=jax.ShapeDtypeStruct((n_e, 32), jnp.float32),
    )(mins, xi, xj, eat, w1i, w1j, w1e, b1, w2, b2)


def _tc_node1(x8, a0, a1, wcx, wca, wcb, wcc, bc, wix, wih, wjx, wjh, n):
    return pl.pallas_call(
        _node1_body,
        grid=(1,),
        in_specs=[_full((n, 8)), _full((n, 32)), _full((n, 32)),
                  _full((8, 16)), _full((16, 16)), _full((16, 16)),
                  _full((16, 16)), _full((1, 16)), _full((8, 32)),
                  _full((16, 32)), _full((8, 32)), _full((16, 32))],
        out_specs=[_full((n, 32)), _full((n, 32))],
        out_shape=[jax.ShapeDtypeStruct((n, 32), jnp.float32),
                   jax.ShapeDtypeStruct((n, 32), jnp.float32)],
    )(x8, a0, a1, wcx, wca, wcb, wcc, bc, wix, wih, wjx, wjh)


def _tc_edge2(gi, gj, eat, w1e, b1, w2, b2, n_e):
    grid = (n_e // _EB,)
    return pl.pallas_call(
        _edge2_body,
        grid=grid,
        in_specs=[_eblk((_EB, 32)), _eblk((_EB, 32)), _teblk((8, _EB)),
                  _full((8, 32)), _full((1, 32)), _full((32, 16)),
                  _full((1, 16))],
        out_specs=_eblk((_EB, 16)),
        out_shape=jax.ShapeDtypeStruct((n_e, 16), jnp.float32),
    )(gi, gj, eat, w1e, b1, w2, b2)


def _tc_node2(x8, g0, g1, wcx, wca, bc, n):
    return pl.pallas_call(
        _node2_body,
        grid=(1,),
        in_specs=[_full((n, 8)), _full((n, 16)), _full((n, 16)),
                  _full((8, 16)), _full((16, 16)), _full((1, 16))],
        out_specs=_full((n, 16)),
        out_shape=jax.ShapeDtypeStruct((n, 16), jnp.float32),
    )(x8, g0, g1, wcx, wca, bc)


def kernel(x, edge_attr, W1a, b1a, W2a, b2a, Wc2, bc2, W1b, b1b, W2b, b2b,
           Wc4, bc4, edge_index):
    n = x.shape[0]
    n_e = edge_index.shape[1]
    src = edge_index[0]
    dst = edge_index[1]

    x8 = jnp.pad(x, ((0, 0), (0, 3)))
    ea8t = jnp.pad(edge_attr.T, ((0, 5), (0, 0)))

    z8 = jnp.zeros((8, 32), jnp.float32)
    w1i = z8.at[0:5].set(W1a[0:5])
    w1j = z8.at[0:5].set(W1a[5:10])
    w1e = z8.at[0:3].set(W1a[10:13])
    b1 = b1a.reshape(1, 32)
    b2 = b2a.reshape(1, 16)

    wcx = jnp.zeros((8, 16), jnp.float32).at[0:5].set(Wc2[0:5])
    wca = Wc2[5:21]
    wcb = Wc2[21:37]
    wcc = Wc2[37:53]
    bc = bc2.reshape(1, 16)

    wix = z8.at[0:5].set(W1b[0:5])
    wih = W1b[5:21]
    wjx = z8.at[0:5].set(W1b[21:26])
    wjh = W1b[26:42]
    w1be = z8.at[0:3].set(W1b[42:45])
    b1l2 = b1b.reshape(1, 32)
    b2l2 = b2b.reshape(1, 16)

    wc4x = jnp.zeros((8, 16), jnp.float32).at[0:5].set(Wc4[0:5])
    wc4a = Wc4[5:21]
    bc4r = bc4.reshape(1, 16)

    xi, xj = _sc_gather_pair(x8, x8, src, dst, 8, _KB)

    mins = _tc_mins(xi, xj, n_e)
    msgax = _tc_edge1(mins, xi, xj, ea8t, w1i, w1j, w1e, b1, W2a, b2, n_e)

    accs = _sc_scatter_add(msgax, dst, jnp.zeros((n, 32), jnp.float32), n, 32)

    p2i, p2j = _tc_node1(x8, accs[0], accs[1], wcx, wca, wcb, wcc, bc,
                         wix, wih, wjx, wjh, n)

    gi, gj = _sc_gather_pair(p2i, p2j, src, dst, 32, 10)

    msg2 = _tc_edge2(gi, gj, ea8t, w1be, b1l2, W2b, b2l2, n_e)

    gs = _sc_scatter_add(msg2, dst, jnp.zeros((n, 16), jnp.float32), n, 16)

    return _tc_node2(x8, gs[0], gs[1], wc4x, wc4a, bc4r, n)


# speedup vs baseline: 6.8486x; 1.0470x over previous
import functools

import jax
import jax.numpy as jnp
from jax import lax
from jax.experimental import pallas as pl
from jax.experimental.pallas import tpu as pltpu
from jax.experimental.pallas import tpu_sc as plsc

_NC = 2
_NS = 16
_NW = _NC * _NS

_OBS = 0.5
_ATT = 0.3
_COMM = 0.7

_EB = 6400


_IB = 80
_KB = 25


def _sc_gather_pair(t_i, t_j, src, dst, w, kb):
    n_e = src.shape[0]
    per_w = n_e // _NW
    c = _IB * kb
    n_chunks = per_w // c
    src2 = src.reshape(n_e // _IB, _IB)
    dst2 = dst.reshape(n_e // _IB, _IB)
    mesh = plsc.VectorSubcoreMesh(core_axis_name="c", subcore_axis_name="s")

    @functools.partial(
        pl.kernel,
        out_type=[jax.ShapeDtypeStruct((n_e, w), jnp.float32),
                  jax.ShapeDtypeStruct((n_e, w), jnp.float32)],
        mesh=mesh,
        compiler_params=pltpu.CompilerParams(use_tc_tiling_on_sc=False),
        scratch_types=[pltpu.VMEM((kb, _IB), jnp.int32),
                       pltpu.VMEM((kb, _IB), jnp.int32),
                       pltpu.VMEM((c, w), jnp.float32),
                       pltpu.VMEM((c, w), jnp.float32),
                       pltpu.SemaphoreType.DMA,
                       pltpu.SemaphoreType.DMA],
    )
    def k(ti_hbm, tj_hbm, src_hbm, dst_hbm, xi_hbm, xj_hbm, dbuf, sbuf,
          xib, xjb, sem1, sem2):
        wid = lax.axis_index("s") * _NC + lax.axis_index("c")
        base = wid * per_w

        def chunk(kk, _):
            off = base + kk * c
            row = off // _IB
            pltpu.sync_copy(dst_hbm.at[pl.ds(row, kb)], dbuf)
            pltpu.sync_copy(src_hbm.at[pl.ds(row, kb)], sbuf)
            for j in range(kb):
                pltpu.async_copy(ti_hbm.at[dbuf.at[j]],
                                 xib.at[pl.ds(j * _IB, _IB)], sem1)
                pltpu.async_copy(tj_hbm.at[sbuf.at[j]],
                                 xjb.at[pl.ds(j * _IB, _IB)], sem2)
            for j in range(kb):
                pltpu.make_async_copy(ti_hbm.at[dbuf.at[j]],
                                      xib.at[pl.ds(j * _IB, _IB)], sem1).wait()
                pltpu.make_async_copy(tj_hbm.at[sbuf.at[j]],
                                      xjb.at[pl.ds(j * _IB, _IB)], sem2).wait()
            pltpu.sync_copy(xib, xi_hbm.at[pl.ds(off, c)])
            pltpu.sync_copy(xjb, xj_hbm.at[pl.ds(off, c)])
            return 0

        lax.fori_loop(0, n_chunks, chunk, 0)

    return k(t_i, t_j, src2, dst2)


def _sc_scatter_add(msg, dst, zeros, n, w):
    n_e = msg.shape[0]
    per_w = n_e // _NW
    c = _IB * _KB
    n_chunks = per_w // c
    rows = n // _NS
    dst2 = dst.reshape(n_e // _IB, _IB)
    mesh = plsc.VectorSubcoreMesh(core_axis_name="c", subcore_axis_name="s")

    @functools.partial(
        pl.kernel,
        out_type=jax.ShapeDtypeStruct((2, n, w), jnp.float32),
        mesh=mesh,
        compiler_params=pltpu.CompilerParams(use_tc_tiling_on_sc=False),
        scratch_types=[pltpu.VMEM((_KB, _IB), jnp.int32),
                       pltpu.VMEM((c, w), jnp.float32),
                       pltpu.VMEM_SHARED((n, w), jnp.float32)],
    )
    def k(msg_hbm, dst_hbm, zeros_hbm, accs_hbm, dbuf, mbuf, acc_sh):
        cid = lax.axis_index("c")
        sid = lax.axis_index("s")
        wid = sid * _NC + cid
        base = wid * per_w
        pltpu.sync_copy(zeros_hbm.at[pl.ds(sid * rows, rows)],
                        acc_sh.at[pl.ds(sid * rows, rows)])
        plsc.subcore_barrier()

        def chunk(kk, _):
            off = base + kk * c
            row = off // _IB
            pltpu.sync_copy(dst_hbm.at[pl.ds(row, _KB)], dbuf)
            pltpu.sync_copy(msg_hbm.at[pl.ds(off, c)], mbuf)
            for j in range(_KB):
                pltpu.sync_copy(mbuf.at[pl.ds(j * _IB, _IB)],
                                acc_sh.at[dbuf.at[j]], add=True)
            return 0

        lax.fori_loop(0, n_chunks, chunk, 0)
        plsc.subcore_barrier()
        pltpu.sync_copy(acc_sh.at[pl.ds(sid * rows, rows)],
                        accs_hbm.at[cid].at[pl.ds(sid * rows, rows)])

    return k(msg, dst2, zeros)


def _min_body(xi_ref, xj_ref, out_ref):
    i = pl.program_id(0)
    ts = xj_ref[:, 0:1]
    td = xi_ref[:, 0:1]
    inf = jnp.float32(jnp.inf)
    a = jnp.min(ts)
    b = jnp.min(jnp.where(td == 1.0, ts, inf))
    c = jnp.min(jnp.where(td == 2.0, ts, inf))
    row = lax.broadcasted_iota(jnp.int32, (8, 128), 0)
    vals = jnp.where(row == 0, a, jnp.where(row == 1, b, jnp.where(row == 2, c, inf)))

    @pl.when(i == 0)
    def _():
        out_ref[...] = vals

    @pl.when(i > 0)
    def _():
        out_ref[...] = jnp.minimum(out_ref[...], vals)


_EAT_DN = (((0,), (0,)), ((), ()))


def _mask_cols(eat, thrs):
    dist = eat[0:1, :]
    rows = jnp.concatenate([(dist < t).astype(jnp.float32) for t in thrs],
                           axis=0)
    k = len(thrs)
    eye = (lax.broadcasted_iota(jnp.int32, (k, k), 0)
           == lax.broadcasted_iota(jnp.int32, (k, k), 1)).astype(jnp.float32)
    return lax.dot_general(rows, eye, _EAT_DN)


def _edge1_body(mins_ref, xi_ref, xj_ref, eat_ref, w1i_ref, w1j_ref, w1e_ref,
                b1_ref, w2_ref, b2_ref, out_ref):
    xi = xi_ref[...]
    xj = xj_ref[...]
    eat = eat_ref[...]
    h = jnp.maximum(
        xi @ w1i_ref[...] + xj @ w1j_ref[...]
        + lax.dot_general(eat, w1e_ref[...], _EAT_DN) + b1_ref[...],
        0.0)
    msg = h @ w2_ref[...] + b2_ref[...]
    mins = mins_ref[...]
    inf = jnp.float32(jnp.inf)

    def thr(m):
        return jnp.where(m == 0.0, _OBS, jnp.where(m == 1.0, _ATT, inf))

    thr_a = thr(mins[0:1, 0:1])
    thr_b = thr(mins[1:2, 0:1])
    thr_c = thr(mins[2:3, 0:1])
    masks = _mask_cols(eat, [thr_a, thr_b, thr_c])
    td = xi[:, 0:1]
    m_a = masks[:, 0:1]
    m_x = jnp.where(td == 1.0, masks[:, 1:2], masks[:, 2:3])
    out_ref[...] = jnp.concatenate([msg * m_a, msg * m_x], axis=1)


def _node1_body(x_ref, *refs):
    (wcx_ref, wca_ref, wcb_ref, wcc_ref, bc_ref, wix_ref, wih_ref, wjx_ref,
     wjh_ref, p2i_ref, p2j_ref) = refs[-11:]
    acc_refs = refs[:-11]
    x = x_ref[...]
    acc = acc_refs[0][...]
    for r in acc_refs[1:]:
        acc = acc + r[...]
    agent = acc[:, 0:16]
    extra = acc[:, 16:32]
    t = x[:, 0:1]
    m1 = (t == 1.0).astype(x.dtype)
    m2 = (t == 2.0).astype(x.dtype)
    rx = jnp.maximum(x, 0.0)
    ra = jnp.maximum(agent, 0.0)
    re = jnp.maximum(extra, 0.0)
    h = (rx @ wcx_ref[...] + ra @ wca_ref[...] + (re * m1) @ wcb_ref[...]
         + (re * m2) @ wcc_ref[...] + bc_ref[...])
    p2i_ref[...] = x @ wix_ref[...] + h @ wih_ref[...]
    p2j_ref[...] = x @ wjx_ref[...] + h @ wjh_ref[...]


def _edge2_body(gi_ref, gj_ref, eat_ref, w1e_ref, b1_ref, w2_ref, b2_ref,
                out_ref):
    eat = eat_ref[...]
    pre = (gi_ref[...] + gj_ref[...]
           + lax.dot_general(eat, w1e_ref[...], _EAT_DN) + b1_ref[...])
    msg = jnp.maximum(pre, 0.0) @ w2_ref[...] + b2_ref[...]
    m = _mask_cols(eat, [jnp.full((1, 1), _COMM, jnp.float32)])
    out_ref[...] = msg * m


def _node2_body(x_ref, *refs):
    wcx_ref, wca_ref, bc_ref, out_ref = refs[-4:]
    g_refs = refs[:-4]
    x = x_ref[...]
    t = x[:, 0:1]
    g = g_refs[0][...]
    for r in g_refs[1:]:
        g = g + r[...]
    ag = g * (t == 0.0).astype(x.dtype)
    out_ref[...] = (jnp.maximum(x, 0.0) @ wcx_ref[...]
                    + jnp.maximum(ag, 0.0) @ wca_ref[...] + bc_ref[...])


def _full(shape):
    return pl.BlockSpec(shape, lambda *_: tuple(0 for _ in shape))


def _eblk(shape):
    return pl.BlockSpec(shape, lambda i: (i, 0))


def _tc_mins(xi, xj, n_e):
    grid = (n_e // _EB,)
    return pl.pallas_call(
        _min_body,
        grid=grid,
        in_specs=[_eblk((_EB, 8)), _eblk((_EB, 8))],
        out_specs=_full((8, 128)),
        out_shape=jax.ShapeDtypeStruct((8, 128), jnp.float32),
    )(xi, xj)


def _teblk(shape):
    return pl.BlockSpec(shape, lambda i: (0, i))


def _tc_edge1(mins, xi, xj, eat, w1i, w1j, w1e, b1, w2, b2, n_e):
    grid = (n_e // _EB,)
    return pl.pallas_call(
        _edge1_body,
        grid=grid,
        in_specs=[_full((8, 128)), _eblk((_EB, 8)), _eblk((_EB, 8)),
                  _teblk((8, _EB)), _full((8, 32)), _full((8, 32)),
                  _full((8, 32)), _full((1, 32)), _full((32, 16)),
                  _full((1, 16))],
        out_specs=_eblk((_EB, 32)),
        out_shape=jax.ShapeDtypeStruct((n_e, 32), jnp.float32),
    )(mins, xi, xj, eat, w1i, w1j, w1e, b1, w2, b2)


def _tc_node1(x8, accs, wcx, wca, wcb, wcc, bc, wix, wih, wjx, wjh, n):
    return pl.pallas_call(
        _node1_body,
        grid=(1,),
        in_specs=([_full((n, 8))] + [_full((n, 32))] * len(accs)
                  + [_full((8, 16)), _full((16, 16)), _full((16, 16)),
                     _full((16, 16)), _full((1, 16)), _full((8, 32)),
                     _full((16, 32)), _full((8, 32)), _full((16, 32))]),
        out_specs=[_full((n, 32)), _full((n, 32))],
        out_shape=[jax.ShapeDtypeStruct((n, 32), jnp.float32),
                   jax.ShapeDtypeStruct((n, 32), jnp.float32)],
    )(x8, *accs, wcx, wca, wcb, wcc, bc, wix, wih, wjx, wjh)


def _tc_edge2(gi, gj, eat, w1e, b1, w2, b2, n_e):
    grid = (n_e // _EB,)
    return pl.pallas_call(
        _edge2_body,
        grid=grid,
        in_specs=[_eblk((_EB, 32)), _eblk((_EB, 32)), _teblk((8, _EB)),
                  _full((8, 32)), _full((1, 32)), _full((32, 16)),
                  _full((1, 16))],
        out_specs=_eblk((_EB, 16)),
        out_shape=jax.ShapeDtypeStruct((n_e, 16), jnp.float32),
    )(gi, gj, eat, w1e, b1, w2, b2)


def _tc_node2(x8, gs, wcx, wca, bc, n):
    return pl.pallas_call(
        _node2_body,
        grid=(1,),
        in_specs=([_full((n, 8))] + [_full((n, 16))] * len(gs)
                  + [_full((8, 16)), _full((16, 16)), _full((1, 16))]),
        out_specs=_full((n, 16)),
        out_shape=jax.ShapeDtypeStruct((n, 16), jnp.float32),
    )(x8, *gs, wcx, wca, bc)


def kernel(x, edge_attr, W1a, b1a, W2a, b2a, Wc2, bc2, W1b, b1b, W2b, b2b,
           Wc4, bc4, edge_index):
    n = x.shape[0]
    n_e = edge_index.shape[1]
    src = edge_index[0]
    dst = edge_index[1]

    x8 = jnp.pad(x, ((0, 0), (0, 3)))
    ea8t = jnp.pad(edge_attr.T, ((0, 5), (0, 0)))

    z8 = jnp.zeros((8, 32), jnp.float32)
    w1i = z8.at[0:5].set(W1a[0:5])
    w1j = z8.at[0:5].set(W1a[5:10])
    w1e = z8.at[0:3].set(W1a[10:13])
    b1 = b1a.reshape(1, 32)
    b2 = b2a.reshape(1, 16)

    wcx = jnp.zeros((8, 16), jnp.float32).at[0:5].set(Wc2[0:5])
    wca = Wc2[5:21]
    wcb = Wc2[21:37]
    wcc = Wc2[37:53]
    bc = bc2.reshape(1, 16)

    wix = z8.at[0:5].set(W1b[0:5])
    wih = W1b[5:21]
    wjx = z8.at[0:5].set(W1b[21:26])
    wjh = W1b[26:42]
    w1be = z8.at[0:3].set(W1b[42:45])
    b1l2 = b1b.reshape(1, 32)
    b2l2 = b2b.reshape(1, 16)

    wc4x = jnp.zeros((8, 16), jnp.float32).at[0:5].set(Wc4[0:5])
    wc4a = Wc4[5:21]
    bc4r = bc4.reshape(1, 16)

    n_h = 2
    e_h = n_e // n_h
    srcs = [src[p * e_h:(p + 1) * e_h] for p in range(n_h)]
    dsts = [dst[p * e_h:(p + 1) * e_h] for p in range(n_h)]
    eats = [ea8t[:, p * e_h:(p + 1) * e_h] for p in range(n_h)]
    z32 = jnp.zeros((n, 32), jnp.float32)
    z16 = jnp.zeros((n, 16), jnp.float32)

    gathered = [_sc_gather_pair(x8, x8, srcs[p], dsts[p], 8, _KB)
                for p in range(n_h)]
    mins_p = [_tc_mins(xi, xj, e_h) for xi, xj in gathered]
    mins = jnp.minimum(mins_p[0], mins_p[1])

    accs = []
    for p in range(n_h):
        xi, xj = gathered[p]
        msgax = _tc_edge1(mins, xi, xj, eats[p], w1i, w1j, w1e, b1, W2a, b2,
                          e_h)
        acc = _sc_scatter_add(msgax, dsts[p], z32, n, 32)
        accs.extend([acc[0], acc[1]])

    p2i, p2j = _tc_node1(x8, accs, wcx, wca, wcb, wcc, bc,
                         wix, wih, wjx, wjh, n)

    gsl = []
    for p in range(n_h):
        gi, gj = _sc_gather_pair(p2i, p2j, srcs[p], dsts[p], 32, 5)
        msg2 = _tc_edge2(gi, gj, eats[p], w1be, b1l2, W2b, b2l2, e_h)
        g = _sc_scatter_add(msg2, dsts[p], z16, n, 16)
        gsl.extend([g[0], g[1]])

    return _tc_node2(x8, gsl, wc4x, wc4a, bc4r, n)
